# baseline (speedup 1.0000x reference)
import jax
import jax.numpy as jnp
from jax.experimental import pallas as pl

_D = 64
_K = 512
_N_ITER = 6
_PERTURB = 1e-05


def _perturbations():
    rng = jax.random.key(1)
    rs, r2s = [], []
    curr, nxt = 1, 2
    while nxt <= _K:
        rng, k1 = jax.random.split(rng)
        rs.append(jax.random.normal(k1, (curr, _D), jnp.float32) * _PERTURB)
        stage_r2 = []
        curr, nxt = nxt, nxt * 2
        for _ in range(_N_ITER):
            rng, k2 = jax.random.split(rng)
            stage_r2.append(jax.random.normal(k2, (curr, _D), jnp.float32) * _PERTURB)
        r2s.append(stage_r2)
    return rs, r2s


def _nearest_body(x_ref, cb_ref, x2_ref, c2_ref, idx_ref):
    x = x_ref[...]
    cb = cb_ref[...]
    n, k = x.shape[0], cb.shape[0]
    x2 = x2_ref[...]
    c2r = c2_ref[...]
    mm = jax.lax.dot_general(x, cb, (((1,), (1,)), ((), ())),
                             preferred_element_type=jnp.float32)
    d2 = x2 + c2r - 2.0 * mm
    mind = jnp.min(d2, axis=1, keepdims=True)
    ilane = jax.lax.broadcasted_iota(jnp.int32, (n, k), 1)
    idx_ref[...] = jnp.min(jnp.where(d2 == mind, ilane, k), axis=1, keepdims=True)


def _nearest2(x, codebook):
    n, k = x.shape[0], codebook.shape[0]
    x2 = jnp.sum(x * x, axis=1, keepdims=True)
    c2 = jnp.sum(codebook * codebook, axis=1)
    idx = pl.pallas_call(
        _nearest_body,
        out_shape=jax.ShapeDtypeStruct((n, 1), jnp.int32),
    )(x, codebook, x2, c2.reshape(1, k))
    return idx.reshape(n)


def _probe_body(x_ref, cb_ref, x2_ref, c2_ref, mm_ref, idx_ref):
    x = x_ref[...]
    cb = cb_ref[...]
    n, k = x.shape[0], cb.shape[0]
    x2 = jnp.sum(x * x, axis=1, keepdims=True)
    c2 = jnp.sum(cb * cb, axis=1, keepdims=True)
    c2r = c2.reshape(1, k)
    mm = jax.lax.dot_general(x, cb, (((1,), (1,)), ((), ())),
                             preferred_element_type=jnp.float32)
    d2 = x2 + c2r - 2.0 * mm
    mind = jnp.min(d2, axis=1, keepdims=True)
    ilane = jax.lax.broadcasted_iota(jnp.int32, (n, k), 1)
    x2_ref[...] = x2
    c2_ref[...] = c2
    mm_ref[...] = mm
    idx_ref[...] = jnp.min(jnp.where(d2 == mind, ilane, k), axis=1, keepdims=True)


def _probe(x, codebook, idx_x, distance):
    n, k = x.shape[0], codebook.shape[0]
    x2p, c2p, mmp, idxp = pl.pallas_call(
        _probe_body,
        out_shape=(jax.ShapeDtypeStruct((n, 1), jnp.float32),
                   jax.ShapeDtypeStruct((k, 1), jnp.float32),
                   jax.ShapeDtypeStruct((n, k), jnp.float32),
                   jax.ShapeDtypeStruct((n, 1), jnp.int32)),
    )(x, codebook)
    x2x = jnp.sum(x * x, axis=1, keepdims=True)
    c2x = jnp.sum(codebook * codebook, axis=1, keepdims=True)
    mmx = x @ codebook.T
    b1 = jnp.any(x2p != x2x).astype(jnp.float32)
    b2 = jnp.any(c2p != c2x).astype(jnp.float32)
    b3 = jnp.any(mmp != mmx).astype(jnp.float32)
    c4 = jnp.sum((idxp.reshape(n) != idx_x).astype(jnp.int32))
    code = (0.7 * b1 + 0.07 * b2 + 0.007 * b3 + 0.0007 * (c4 > 0)
            + 7e-5 * (c4 >= 10) + 7e-6 * (c4 >= 100))
    return distance * (1.0 + code)


def kernel(x):
    x = x.reshape(-1, _D)
    n = x.shape[0]
    rs, r2s = _perturbations()
    codebook = jnp.full((_K, _D), 1e10, dtype=x.dtype)
    codebook = codebook.at[0].set(jnp.mean(x, axis=0))
    distance = jnp.asarray(0.0, dtype=x.dtype)
    curr, nxt = 1, 2
    s = 0
    while nxt <= _K:
        cb = codebook[:curr]
        r = rs[s]
        codebook = codebook.at[curr:nxt].set(cb - r)
        codebook = codebook.at[:curr].add(r)
        curr, nxt = nxt, nxt * 2
        for i in range(_N_ITER):
            idx = _nearest2(x, codebook)
            xq = codebook[idx]
            distance = jnp.sum((x - xq) ** 2) / n
            n_data = jnp.bincount(idx, length=curr).astype(x.dtype)
            mask = n_data >= 1
            cent = jax.ops.segment_sum(x, idx, num_segments=curr)
            safe = jnp.maximum(n_data, 1.0)
            cent = jnp.where(mask[:, None], cent / safe[:, None], cent)
            m = jnp.argmax(n_data)
            r2 = r2s[s][i]
            bad = ~mask
            n_bad = jnp.sum(bad.astype(x.dtype))
            copied = jnp.broadcast_to(cent[m], (curr, _D))
            cent = jnp.where(bad[:, None], copied - r2, cent)
            r_mean = jnp.sum(r2 * bad[:, None].astype(x.dtype), axis=0) / jnp.maximum(n_bad, 1.0)
            cent = cent.at[m].add(jnp.where(n_bad > 0, r_mean, jnp.zeros_like(r_mean)))
            codebook = codebook.at[:curr].set(cent)
        s += 1
    idx = _nearest2(x, codebook)
    return codebook, idx, distance


# speedup vs baseline: 1.1684x; 1.1684x over previous
import jax
import jax.numpy as jnp
from jax.experimental import pallas as pl

_D = 64
_K = 512
_N_ITER = 6
_PERTURB = 1e-05


def _perturbations():
    rng = jax.random.key(1)
    rs, r2s = [], []
    curr, nxt = 1, 2
    while nxt <= _K:
        rng, k1 = jax.random.split(rng)
        rs.append(jax.random.normal(k1, (curr, _D), jnp.float32) * _PERTURB)
        stage_r2 = []
        curr, nxt = nxt, nxt * 2
        for _ in range(_N_ITER):
            rng, k2 = jax.random.split(rng)
            stage_r2.append(jax.random.normal(k2, (curr, _D), jnp.float32) * _PERTURB)
        r2s.append(stage_r2)
    return rs, r2s


def _nearest_body(x_ref, cb_ref, x2_ref, c2_ref, idx_ref, cnt_ref):
    x = x_ref[...]
    cb = cb_ref[...]
    n, k = x.shape[0], cb.shape[0]
    x2 = x2_ref[...]
    c2r = c2_ref[...]
    mm = jax.lax.dot_general(x, cb, (((1,), (1,)), ((), ())),
                             preferred_element_type=jnp.float32)
    d2 = x2 + c2r - 2.0 * mm
    mind = jnp.min(d2, axis=1, keepdims=True)
    ilane = jax.lax.broadcasted_iota(jnp.int32, (n, k), 1)
    idx2d = jnp.min(jnp.where(d2 == mind, ilane, k), axis=1, keepdims=True)
    idx_ref[...] = idx2d
    onehot = (ilane == idx2d).astype(jnp.float32)
    cnt_ref[...] = jnp.sum(onehot, axis=0, keepdims=True)


def _nearest2(x, codebook):
    n, k = x.shape[0], codebook.shape[0]
    x2 = jnp.sum(x * x, axis=1, keepdims=True)
    c2 = jnp.sum(codebook * codebook, axis=1)
    idx, cnt = pl.pallas_call(
        _nearest_body,
        out_shape=(jax.ShapeDtypeStruct((n, 1), jnp.int32),
                   jax.ShapeDtypeStruct((1, k), jnp.float32)),
    )(x, codebook, x2, c2.reshape(1, k))
    return idx.reshape(n), cnt.reshape(k)


def _probe_body(x_ref, cb_ref, x2_ref, c2_ref, mm_ref, idx_ref):
    x = x_ref[...]
    cb = cb_ref[...]
    n, k = x.shape[0], cb.shape[0]
    x2 = jnp.sum(x * x, axis=1, keepdims=True)
    c2 = jnp.sum(cb * cb, axis=1, keepdims=True)
    c2r = c2.reshape(1, k)
    mm = jax.lax.dot_general(x, cb, (((1,), (1,)), ((), ())),
                             preferred_element_type=jnp.float32)
    d2 = x2 + c2r - 2.0 * mm
    mind = jnp.min(d2, axis=1, keepdims=True)
    ilane = jax.lax.broadcasted_iota(jnp.int32, (n, k), 1)
    x2_ref[...] = x2
    c2_ref[...] = c2
    mm_ref[...] = mm
    idx_ref[...] = jnp.min(jnp.where(d2 == mind, ilane, k), axis=1, keepdims=True)


def _probe(x, codebook, idx_x, distance):
    n, k = x.shape[0], codebook.shape[0]
    x2p, c2p, mmp, idxp = pl.pallas_call(
        _probe_body,
        out_shape=(jax.ShapeDtypeStruct((n, 1), jnp.float32),
                   jax.ShapeDtypeStruct((k, 1), jnp.float32),
                   jax.ShapeDtypeStruct((n, k), jnp.float32),
                   jax.ShapeDtypeStruct((n, 1), jnp.int32)),
    )(x, codebook)
    x2x = jnp.sum(x * x, axis=1, keepdims=True)
    c2x = jnp.sum(codebook * codebook, axis=1, keepdims=True)
    mmx = x @ codebook.T
    b1 = jnp.any(x2p != x2x).astype(jnp.float32)
    b2 = jnp.any(c2p != c2x).astype(jnp.float32)
    b3 = jnp.any(mmp != mmx).astype(jnp.float32)
    c4 = jnp.sum((idxp.reshape(n) != idx_x).astype(jnp.int32))
    code = (0.7 * b1 + 0.07 * b2 + 0.007 * b3 + 0.0007 * (c4 > 0)
            + 7e-5 * (c4 >= 10) + 7e-6 * (c4 >= 100))
    return distance * (1.0 + code)


def kernel(x):
    x = x.reshape(-1, _D)
    n = x.shape[0]
    rs, r2s = _perturbations()
    codebook = jnp.full((_K, _D), 1e10, dtype=x.dtype)
    codebook = codebook.at[0].set(jnp.mean(x, axis=0))
    distance = jnp.asarray(0.0, dtype=x.dtype)
    curr, nxt = 1, 2
    s = 0
    while nxt <= _K:
        cb = codebook[:curr]
        r = rs[s]
        codebook = codebook.at[curr:nxt].set(cb - r)
        codebook = codebook.at[:curr].add(r)
        curr, nxt = nxt, nxt * 2
        for i in range(_N_ITER):
            idx, cnt = _nearest2(x, codebook)
            xq = codebook[idx]
            distance = jnp.sum((x - xq) ** 2) / n
            n_data = cnt[:curr]
            mask = n_data >= 1
            cent = jax.ops.segment_sum(x, idx, num_segments=curr)
            safe = jnp.maximum(n_data, 1.0)
            cent = jnp.where(mask[:, None], cent / safe[:, None], cent)
            m = jnp.argmax(n_data)
            r2 = r2s[s][i]
            bad = ~mask
            n_bad = jnp.sum(bad.astype(x.dtype))
            copied = jnp.broadcast_to(cent[m], (curr, _D))
            cent = jnp.where(bad[:, None], copied - r2, cent)
            r_mean = jnp.sum(r2 * bad[:, None].astype(x.dtype), axis=0) / jnp.maximum(n_bad, 1.0)
            cent = cent.at[m].add(jnp.where(n_bad > 0, r_mean, jnp.zeros_like(r_mean)))
            codebook = codebook.at[:curr].set(cent)
        s += 1
    idx, _ = _nearest2(x, codebook)
    return codebook, idx, distance
